# Initial kernel scaffold; baseline (speedup 1.0000x reference)
import jax
import jax.numpy as jnp
from jax.experimental import pallas as pl


def kernel(k_masks, fn, mean):
    raise NotImplementedError("write your pallas kernel here")


# speedup vs baseline: 5.4824x; 5.4824x over previous
import functools

import jax
import jax.numpy as jnp
from jax import lax
from jax.experimental import pallas as pl
from jax.experimental.pallas import tpu as pltpu
from jax.experimental.pallas import tpu_sc as plsc

L = 32
N = 32768
LANES = 16
CH = N // LANES
NBINS = 256
HIST = NBINS * LANES


def _sort_scatter_body(km_hbm, fn_hbm, out_hbm, keys, pay_a, pay_b, hist):
    wid = lax.axis_index("c") * 16 + lax.axis_index("s")
    lane = lax.iota(jnp.int32, 16)

    pltpu.sync_copy(fn_hbm.at[wid], keys)

    @plsc.parallel_loop(0, CH, unroll=4)
    def _transform(i):
        b = keys[pl.ds(i * 16, 16)]
        keys[pl.ds(i * 16, 16)] = jnp.where(b >= 0, b ^ 0x7FFFFFFF, b)

    bufs = (pay_a, pay_b)
    for p in range(4):
        shift = 8 * p
        src = bufs[p % 2]
        dst = bufs[(p + 1) % 2]

        @plsc.parallel_loop(0, NBINS, unroll=4)
        def _zero(v):
            hist[pl.ds(v * 16, 16)] = jnp.zeros((16,), jnp.int32)

        def _count(i, carry, p=p, shift=shift, src=src):
            if p == 0:
                j = lane * CH + i
            else:
                j = src[pl.ds(i * 16, 16)] & 0x7FFF
            k = plsc.load_gather(keys, [j])
            d = lax.shift_right_logical(k, shift) & (NBINS - 1)
            cidx = d * 16 + lane
            c = plsc.load_gather(hist, [cidx])
            plsc.store_scatter(hist, [cidx], c + 1)
            src[pl.ds(i * 16, 16)] = (c << 15) | j
            return carry

        lax.fori_loop(0, CH, _count, jnp.int32(0))

        def _prefix(v, carry):
            x = hist[pl.ds(v * 16, 16)]
            s = plsc.cumsum(x)
            hist[pl.ds(v * 16, 16)] = s - x + carry
            return carry + jnp.sum(x)

        lax.fori_loop(0, NBINS, _prefix, jnp.int32(0))

        @plsc.parallel_loop(0, CH, unroll=4)
        def _permute(i, shift=shift, src=src, dst=dst):
            w = src[pl.ds(i * 16, 16)]
            j = w & 0x7FFF
            occ = lax.shift_right_logical(w, 15)
            k = plsc.load_gather(keys, [j])
            d = lax.shift_right_logical(k, shift) & (NBINS - 1)
            base = plsc.load_gather(hist, [d * 16 + lane])
            rho = base + occ
            addr = (rho & (CH - 1)) * 16 + lax.shift_right_logical(rho, 11)
            plsc.store_scatter(dst, [addr], j)

    pltpu.sync_copy(km_hbm.at[wid], pay_b)

    @plsc.parallel_loop(0, CH, unroll=4)
    def _scatter(i):
        j = pay_a[pl.ds(i * 16, 16)]
        r = lane * CH + i
        v = plsc.load_gather(pay_b, [r])
        plsc.store_scatter(keys, [j], v)

    pltpu.sync_copy(keys, out_hbm.at[wid])


@jax.jit
def _sc_sort_scatter(km_i, fn_i):
    mesh = plsc.VectorSubcoreMesh(core_axis_name="c", subcore_axis_name="s")
    f = pl.kernel(
        _sort_scatter_body,
        out_type=jax.ShapeDtypeStruct((L, N), jnp.int32),
        mesh=mesh,
        compiler_params=pltpu.CompilerParams(needs_layout_passes=False),
        scratch_types=[
            pltpu.VMEM((N,), jnp.int32),
            pltpu.VMEM((N,), jnp.int32),
            pltpu.VMEM((N,), jnp.int32),
            pltpu.VMEM((HIST,), jnp.int32),
        ],
    )
    return f(km_i, fn_i)


def _sigmoid_body(x_ref, m_ref, o_ref):
    t = (x_ref[...] - m_ref[0, 0]) / 0.7
    o_ref[...] = 1.0 / (1.0 + jnp.exp(-t))


@jax.jit
def _tc_sigmoid(fn, mean):
    return pl.pallas_call(
        _sigmoid_body,
        out_shape=jax.ShapeDtypeStruct((L, N), jnp.float32),
        in_specs=[
            pl.BlockSpec((L, N), lambda: (0, 0)),
            pl.BlockSpec(memory_space=pltpu.SMEM),
        ],
        out_specs=pl.BlockSpec((L, N), lambda: (0, 0)),
    )(fn, jnp.reshape(mean, (1, 1)))


def kernel(k_masks, fn, mean):
    km_i = lax.bitcast_convert_type(k_masks, jnp.int32)
    fn_i = lax.bitcast_convert_type(fn, jnp.int32)
    c_i = _sc_sort_scatter(km_i, fn_i)
    ori_masks = lax.bitcast_convert_type(c_i, jnp.float32)
    cout = _tc_sigmoid(fn, mean)
    return ori_masks, cout


# speedup vs baseline: 5.5578x; 1.0137x over previous
import functools

import jax
import jax.numpy as jnp
from jax import lax
from jax.experimental import pallas as pl
from jax.experimental.pallas import tpu as pltpu
from jax.experimental.pallas import tpu_sc as plsc

L = 32
N = 32768
LANES = 16
CH = N // LANES
NBINS = 256
NSTREAM = 4
SCH = CH // NSTREAM


def _sort_scatter_body(km_hbm, fn_hbm, out_hbm, keys, pay_a, pay_b, *hists):
    wid = lax.axis_index("c") * 16 + lax.axis_index("s")
    lane = lax.iota(jnp.int32, 16)

    pltpu.sync_copy(fn_hbm.at[wid], keys)

    @plsc.parallel_loop(0, CH, unroll=8)
    def _transform(i):
        b = keys[pl.ds(i * 16, 16)]
        keys[pl.ds(i * 16, 16)] = jnp.where(b >= 0, b ^ 0x7FFFFFFF, b)

    bufs = (pay_a, pay_b)
    for p in range(4):
        shift = 8 * p
        src = bufs[p % 2]
        dst = bufs[(p + 1) % 2]

        @plsc.parallel_loop(0, NBINS, unroll=8)
        def _zero(v):
            for h in hists:
                h[pl.ds(v * 16, 16)] = jnp.zeros((16,), jnp.int32)

        def _count(i, carry, p=p, shift=shift, src=src):
            for u in range(NSTREAM):
                iu = i + u * SCH
                if p == 0:
                    j = lane * CH + iu
                else:
                    j = src[pl.ds(iu * 16, 16)] & 0x7FFF
                k = plsc.load_gather(keys, [j])
                d = lax.shift_right_logical(k, shift) & (NBINS - 1)
                cidx = d * 16 + lane
                c = plsc.load_gather(hists[u], [cidx])
                plsc.store_scatter(hists[u], [cidx], c + 1)
                src[pl.ds(iu * 16, 16)] = (c << 15) | j
            return carry

        lax.fori_loop(0, SCH, _count, jnp.int32(0))

        def _prefix(v, carry):
            xs = [hists[u][pl.ds(v * 16, 16)] for u in range(NSTREAM)]
            s = xs[0]
            for u in range(1, NSTREAM):
                s = s + xs[u]
            base = plsc.cumsum(s) - s + carry
            for u in range(NSTREAM):
                hists[u][pl.ds(v * 16, 16)] = base
                base = base + xs[u]
            return carry + jnp.sum(s)

        lax.fori_loop(0, NBINS, _prefix, jnp.int32(0))

        for u in range(NSTREAM):

            @plsc.parallel_loop(0, SCH, unroll=8)
            def _permute(i, shift=shift, src=src, dst=dst, u=u):
                iu = i + u * SCH
                w = src[pl.ds(iu * 16, 16)]
                j = w & 0x7FFF
                occ = lax.shift_right_logical(w, 15)
                k = plsc.load_gather(keys, [j])
                d = lax.shift_right_logical(k, shift) & (NBINS - 1)
                base = plsc.load_gather(hists[u], [d * 16 + lane])
                rho = base + occ
                addr = (rho & (CH - 1)) * 16 + lax.shift_right_logical(rho, 11)
                plsc.store_scatter(dst, [addr], j)

    pltpu.sync_copy(km_hbm.at[wid], pay_b)

    @plsc.parallel_loop(0, CH, unroll=8)
    def _scatter(i):
        j = pay_a[pl.ds(i * 16, 16)]
        r = lane * CH + i
        v = plsc.load_gather(pay_b, [r])
        plsc.store_scatter(keys, [j], v)

    pltpu.sync_copy(keys, out_hbm.at[wid])


@jax.jit
def _sc_sort_scatter(km_i, fn_i):
    mesh = plsc.VectorSubcoreMesh(core_axis_name="c", subcore_axis_name="s")
    f = pl.kernel(
        _sort_scatter_body,
        out_type=jax.ShapeDtypeStruct((L, N), jnp.int32),
        mesh=mesh,
        compiler_params=pltpu.CompilerParams(needs_layout_passes=False),
        scratch_types=[
            pltpu.VMEM((N,), jnp.int32),
            pltpu.VMEM((N,), jnp.int32),
            pltpu.VMEM((N,), jnp.int32),
        ] + [pltpu.VMEM((NBINS * 16,), jnp.int32)] * NSTREAM,
    )
    return f(km_i, fn_i)


def _sigmoid_body(x_ref, m_ref, o_ref):
    t = (x_ref[...] - m_ref[0, 0]) / 0.7
    o_ref[...] = 1.0 / (1.0 + jnp.exp(-t))


@jax.jit
def _tc_sigmoid(fn, mean):
    return pl.pallas_call(
        _sigmoid_body,
        out_shape=jax.ShapeDtypeStruct((L, N), jnp.float32),
        in_specs=[
            pl.BlockSpec((L, N), lambda: (0, 0)),
            pl.BlockSpec(memory_space=pltpu.SMEM),
        ],
        out_specs=pl.BlockSpec((L, N), lambda: (0, 0)),
    )(fn, jnp.reshape(mean, (1, 1)))


def kernel(k_masks, fn, mean):
    km_i = lax.bitcast_convert_type(k_masks, jnp.int32)
    fn_i = lax.bitcast_convert_type(fn, jnp.int32)
    c_i = _sc_sort_scatter(km_i, fn_i)
    ori_masks = lax.bitcast_convert_type(c_i, jnp.float32)
    cout = _tc_sigmoid(fn, mean)
    return ori_masks, cout


# speedup vs baseline: 8.8586x; 1.5939x over previous
import functools

import jax
import jax.numpy as jnp
from jax import lax
from jax.experimental import pallas as pl
from jax.experimental.pallas import tpu as pltpu
from jax.experimental.pallas import tpu_sc as plsc

L = 32
N = 32768
LANES = 16
CH = N // LANES
NBINS = 256
NSTREAM = 4
SCH = CH // NSTREAM


def _sort_scatter_body(km_hbm, fn_hbm, out_hbm, keys, pay_a, pay_b, *hists):
    wid = lax.axis_index("c") * 16 + lax.axis_index("s")
    lane = lax.iota(jnp.int32, 16)

    pltpu.sync_copy(fn_hbm.at[wid], keys)

    @plsc.parallel_loop(0, CH, unroll=8)
    def _transform(i):
        b = keys[pl.ds(i * 16, 16)]
        keys[pl.ds(i * 16, 16)] = jnp.where(b >= 0, b ^ 0x7FFFFFFF, b)

    for p in range(4):
        shift = 8 * p

        @plsc.parallel_loop(0, NBINS, unroll=8)
        def _zero(v):
            for h in hists:
                h[pl.ds(v * 16, 16)] = jnp.zeros((16,), jnp.int32)

        def _count(i, carry, p=p, shift=shift):
            if p == 0:
                js = [lane * CH + (i + u * SCH) for u in range(NSTREAM)]
            else:
                js = [pay_a[pl.ds((i + u * SCH) * 16, 16)]
                      for u in range(NSTREAM)]
            ks = [plsc.load_gather(keys, [j]) for j in js]
            cidxs = [(lax.shift_right_logical(k, shift) & (NBINS - 1)) * 16
                     + lane for k in ks]
            cs = [plsc.load_gather(hists[u], [cidxs[u]])
                  for u in range(NSTREAM)]
            for u in range(NSTREAM):
                plsc.store_scatter(hists[u], [cidxs[u]], cs[u] + 1)
            for u in range(NSTREAM):
                pay_b[pl.ds((i + u * SCH) * 16, 16)] = (cs[u] << 15) | js[u]
            return carry

        lax.fori_loop(0, SCH, _count, jnp.int32(0))

        def _prefix(v, carry):
            xs = [hists[u][pl.ds(v * 16, 16)] for u in range(NSTREAM)]
            s = xs[0]
            for u in range(1, NSTREAM):
                s = s + xs[u]
            base = plsc.cumsum(s) - s + carry
            for u in range(NSTREAM):
                hists[u][pl.ds(v * 16, 16)] = base
                base = base + xs[u]
            return carry + jnp.sum(s)

        lax.fori_loop(0, NBINS, _prefix, jnp.int32(0))

        @plsc.parallel_loop(0, SCH, unroll=2)
        def _permute(i, shift=shift):
            ws = [pay_b[pl.ds((i + u * SCH) * 16, 16)]
                  for u in range(NSTREAM)]
            js = [w & 0x7FFF for w in ws]
            occs = [lax.shift_right_logical(w, 15) for w in ws]
            ks = [plsc.load_gather(keys, [j]) for j in js]
            bases = [
                plsc.load_gather(
                    hists[u],
                    [(lax.shift_right_logical(ks[u], shift) & (NBINS - 1))
                     * 16 + lane])
                for u in range(NSTREAM)
            ]
            for u in range(NSTREAM):
                rho = bases[u] + occs[u]
                addr = (rho & (CH - 1)) * 16 + lax.shift_right_logical(rho, 11)
                plsc.store_scatter(pay_a, [addr], js[u])

    pltpu.sync_copy(km_hbm.at[wid], pay_b)

    @plsc.parallel_loop(0, CH, unroll=8)
    def _scatter(i):
        j = pay_a[pl.ds(i * 16, 16)]
        r = lane * CH + i
        v = plsc.load_gather(pay_b, [r])
        plsc.store_scatter(keys, [j], v)

    pltpu.sync_copy(keys, out_hbm.at[wid])


@jax.jit
def _sc_sort_scatter(km_i, fn_i):
    mesh = plsc.VectorSubcoreMesh(core_axis_name="c", subcore_axis_name="s")
    f = pl.kernel(
        _sort_scatter_body,
        out_type=jax.ShapeDtypeStruct((L, N), jnp.int32),
        mesh=mesh,
        compiler_params=pltpu.CompilerParams(needs_layout_passes=False),
        scratch_types=[
            pltpu.VMEM((N,), jnp.int32),
            pltpu.VMEM((N,), jnp.int32),
            pltpu.VMEM((N,), jnp.int32),
        ] + [pltpu.VMEM((NBINS * 16,), jnp.int32)] * NSTREAM,
    )
    return f(km_i, fn_i)


def _sigmoid_body(x_ref, m_ref, o_ref):
    t = (x_ref[...] - m_ref[0, 0]) / 0.7
    o_ref[...] = 1.0 / (1.0 + jnp.exp(-t))


@jax.jit
def _tc_sigmoid(fn, mean):
    return pl.pallas_call(
        _sigmoid_body,
        out_shape=jax.ShapeDtypeStruct((L, N), jnp.float32),
        in_specs=[
            pl.BlockSpec((L, N), lambda: (0, 0)),
            pl.BlockSpec(memory_space=pltpu.SMEM),
        ],
        out_specs=pl.BlockSpec((L, N), lambda: (0, 0)),
    )(fn, jnp.reshape(mean, (1, 1)))


def kernel(k_masks, fn, mean):
    km_i = lax.bitcast_convert_type(k_masks, jnp.int32)
    fn_i = lax.bitcast_convert_type(fn, jnp.int32)
    c_i = _sc_sort_scatter(km_i, fn_i)
    ori_masks = lax.bitcast_convert_type(c_i, jnp.float32)
    cout = _tc_sigmoid(fn, mean)
    return ori_masks, cout


# speedup vs baseline: 17.2891x; 1.9517x over previous
import functools

import jax
import jax.numpy as jnp
from jax import lax
from jax.experimental import pallas as pl
from jax.experimental.pallas import tpu as pltpu
from jax.experimental.pallas import tpu_sc as plsc

L = 32
N = 32768
LANES = 16
NSTREAM = 8
SSZ = N // NSTREAM
SCH = SSZ // LANES
BITS = (11, 11, 10)
SHIFTS = (0, 11, 22)
NB = 2048


def _sort_scatter_body(km_hbm, fn_hbm, out_hbm, keys, pay_a, pay_b, *hists):
    wid = lax.axis_index("c") * 16 + lax.axis_index("s")
    lane = lax.iota(jnp.int32, 16)

    pltpu.sync_copy(fn_hbm.at[wid], keys)

    for p in range(3):
        shift = SHIFTS[p]
        mask = (1 << BITS[p]) - 1

        @plsc.parallel_loop(0, (mask + 1) // 16, unroll=8)
        def _zero(v):
            for h in hists:
                h[pl.ds(v * 16, 16)] = jnp.zeros((16,), jnp.int32)

        def _count(i, carry, p=p, shift=shift, mask=mask):
            if p == 0:
                js = [16 * i + lane + u * SSZ for u in range(NSTREAM)]
                ks = []
                for u in range(NSTREAM):
                    b = keys[pl.ds((i + u * SCH) * 16, 16)]
                    m = jnp.where(b >= 0, b ^ 0x7FFFFFFF, b)
                    keys[pl.ds((i + u * SCH) * 16, 16)] = m
                    ks.append(m)
            else:
                js = [pay_a[pl.ds((i + u * SCH) * 16, 16)] & (N - 1)
                      for u in range(NSTREAM)]
                ks = [plsc.load_gather(keys, [j]) for j in js]
            ds = [lax.shift_right_logical(k, shift) & mask for k in ks]
            sc = [plsc.scan_count(d) for d in ds]
            cs = [plsc.load_gather(hists[u], [ds[u]])
                  for u in range(NSTREAM)]
            tots = [cs[u] + sc[u][0] for u in range(NSTREAM)]
            occs = [t - 1 for t in tots]
            for u in range(NSTREAM):
                plsc.store_scatter(hists[u], [ds[u]], tots[u],
                                   mask=sc[u][1])
            for u in range(NSTREAM):
                pay_b[pl.ds((i + u * SCH) * 16, 16)] = (occs[u] << 15) | js[u]
            return carry

        lax.fori_loop(0, SCH, _count, jnp.int32(0))

        def _prefix(v, carry):
            xs = [hists[u][pl.ds(v * 16, 16)] for u in range(NSTREAM)]
            s = xs[0]
            for u in range(1, NSTREAM):
                s = s + xs[u]
            base = plsc.cumsum(s) - s + carry
            for u in range(NSTREAM):
                hists[u][pl.ds(v * 16, 16)] = base
                base = base + xs[u]
            return carry + jnp.sum(s)

        lax.fori_loop(0, (mask + 1) // 16, _prefix, jnp.int32(0))

        @plsc.parallel_loop(0, SCH, unroll=2)
        def _permute(i, shift=shift, mask=mask):
            ws = [pay_b[pl.ds((i + u * SCH) * 16, 16)]
                  for u in range(NSTREAM)]
            js = [w & 0x7FFF for w in ws]
            occs = [lax.shift_right_logical(w, 15) for w in ws]
            ks = [plsc.load_gather(keys, [j]) for j in js]
            bases = [
                plsc.load_gather(
                    hists[u],
                    [lax.shift_right_logical(ks[u], shift) & mask])
                for u in range(NSTREAM)
            ]
            for u in range(NSTREAM):
                plsc.store_scatter(pay_a, [(bases[u] + occs[u]) & (N - 1)],
                                   js[u])

    pltpu.sync_copy(km_hbm.at[wid], pay_b)

    @plsc.parallel_loop(0, N // 16, unroll=8)
    def _scatter(i):
        j = pay_a[pl.ds(i * 16, 16)] & (N - 1)
        v = pay_b[pl.ds(i * 16, 16)]
        plsc.store_scatter(keys, [j], v)

    pltpu.sync_copy(keys, out_hbm.at[wid])


@jax.jit
def _sc_sort_scatter(km_i, fn_i):
    mesh = plsc.VectorSubcoreMesh(core_axis_name="c", subcore_axis_name="s")
    f = pl.kernel(
        _sort_scatter_body,
        out_type=jax.ShapeDtypeStruct((L, N), jnp.int32),
        mesh=mesh,
        compiler_params=pltpu.CompilerParams(needs_layout_passes=False),
        scratch_types=[
            pltpu.VMEM((N,), jnp.int32),
            pltpu.VMEM((N,), jnp.int32),
            pltpu.VMEM((N,), jnp.int32),
        ] + [pltpu.VMEM((NB,), jnp.int32)] * NSTREAM,
    )
    return f(km_i, fn_i)


def _sigmoid_body(x_ref, m_ref, o_ref):
    t = (x_ref[...] - m_ref[0, 0]) / 0.7
    o_ref[...] = 1.0 / (1.0 + jnp.exp(-t))


@jax.jit
def _tc_sigmoid(fn, mean):
    return pl.pallas_call(
        _sigmoid_body,
        out_shape=jax.ShapeDtypeStruct((L, N), jnp.float32),
        in_specs=[
            pl.BlockSpec((L, N), lambda: (0, 0)),
            pl.BlockSpec(memory_space=pltpu.SMEM),
        ],
        out_specs=pl.BlockSpec((L, N), lambda: (0, 0)),
    )(fn, jnp.reshape(mean, (1, 1)))


def kernel(k_masks, fn, mean):
    km_i = lax.bitcast_convert_type(k_masks, jnp.int32)
    fn_i = lax.bitcast_convert_type(fn, jnp.int32)
    c_i = _sc_sort_scatter(km_i, fn_i)
    ori_masks = lax.bitcast_convert_type(c_i, jnp.float32)
    cout = _tc_sigmoid(fn, mean)
    return ori_masks, cout


# speedup vs baseline: 19.0400x; 1.1013x over previous
import functools

import jax
import jax.numpy as jnp
from jax import lax
from jax.experimental import pallas as pl
from jax.experimental.pallas import tpu as pltpu
from jax.experimental.pallas import tpu_sc as plsc

L = 32
N = 32768
LANES = 16
NSTREAM = 8
SSZ = N // NSTREAM
SCH = SSZ // LANES
BITS = (11, 11, 10)
SHIFTS = (0, 11, 22)
NB = 2048


def _i32(x):
    return plsc.bitcast(x, jnp.int32)


def _f32(x):
    return plsc.bitcast(x, jnp.float32)


def _sort_scatter_body(km_hbm, fn_hbm, out_hbm, keys, pay_a, pay_b, *hists):
    wid = lax.axis_index("c") * 16 + lax.axis_index("s")
    lane = lax.iota(jnp.int32, 16)

    pltpu.sync_copy(fn_hbm.at[wid], keys)

    for p in range(3):
        shift = SHIFTS[p]
        mask = (1 << BITS[p]) - 1

        @plsc.parallel_loop(0, (mask + 1) // 16, unroll=8)
        def _zero(v):
            for h in hists:
                h[pl.ds(v * 16, 16)] = jnp.zeros((16,), jnp.int32)

        def _count(i, carry, p=p, shift=shift, mask=mask):
            if p == 0:
                js = [16 * i + lane + u * SSZ for u in range(NSTREAM)]
                ks = []
                for u in range(NSTREAM):
                    b = _i32(keys[pl.ds((i + u * SCH) * 16, 16)])
                    m = jnp.where(b >= 0, b ^ 0x7FFFFFFF, b)
                    keys[pl.ds((i + u * SCH) * 16, 16)] = _f32(m)
                    ks.append(m)
            else:
                js = [pay_a[pl.ds((i + u * SCH) * 16, 16)] & (N - 1)
                      for u in range(NSTREAM)]
                ks = [_i32(plsc.load_gather(keys, [j])) for j in js]
            ds = [lax.shift_right_logical(k, shift) & mask for k in ks]
            sc = [plsc.scan_count(d) for d in ds]
            cs = [plsc.load_gather(hists[u], [ds[u]])
                  for u in range(NSTREAM)]
            tots = [cs[u] + sc[u][0] for u in range(NSTREAM)]
            occs = [t - 1 for t in tots]
            for u in range(NSTREAM):
                plsc.store_scatter(hists[u], [ds[u]], tots[u],
                                   mask=sc[u][1])
            for u in range(NSTREAM):
                pay_b[pl.ds((i + u * SCH) * 16, 16)] = _f32(
                    (occs[u] << 15) | js[u])
            return carry

        lax.fori_loop(0, SCH, _count, jnp.int32(0))

        def _prefix(v, carry):
            xs = [hists[u][pl.ds(v * 16, 16)] for u in range(NSTREAM)]
            s = xs[0]
            for u in range(1, NSTREAM):
                s = s + xs[u]
            base = plsc.cumsum(s) - s + carry
            for u in range(NSTREAM):
                hists[u][pl.ds(v * 16, 16)] = base
                base = base + xs[u]
            return carry + jnp.sum(s)

        lax.fori_loop(0, (mask + 1) // 16, _prefix, jnp.int32(0))

        @plsc.parallel_loop(0, SCH, unroll=2)
        def _permute(i, shift=shift, mask=mask):
            ws = [_i32(pay_b[pl.ds((i + u * SCH) * 16, 16)])
                  for u in range(NSTREAM)]
            js = [w & 0x7FFF for w in ws]
            occs = [lax.shift_right_logical(w, 15) for w in ws]
            ks = [_i32(plsc.load_gather(keys, [j])) for j in js]
            bases = [
                plsc.load_gather(
                    hists[u],
                    [lax.shift_right_logical(ks[u], shift) & mask])
                for u in range(NSTREAM)
            ]
            for u in range(NSTREAM):
                plsc.store_scatter(pay_a, [(bases[u] + occs[u]) & (N - 1)],
                                   js[u])

    pltpu.sync_copy(km_hbm.at[wid], pay_b)

    @plsc.parallel_loop(0, N // 16, unroll=8)
    def _scatter(i):
        j = pay_a[pl.ds(i * 16, 16)] & (N - 1)
        v = pay_b[pl.ds(i * 16, 16)]
        plsc.store_scatter(keys, [j], v)

    pltpu.sync_copy(keys, out_hbm.at[wid])


@jax.jit
def _sc_sort_scatter(km, fn):
    mesh = plsc.VectorSubcoreMesh(core_axis_name="c", subcore_axis_name="s")
    f = pl.kernel(
        _sort_scatter_body,
        out_type=jax.ShapeDtypeStruct((L, N), jnp.float32),
        mesh=mesh,
        compiler_params=pltpu.CompilerParams(needs_layout_passes=False),
        scratch_types=[
            pltpu.VMEM((N,), jnp.float32),
            pltpu.VMEM((N,), jnp.int32),
            pltpu.VMEM((N,), jnp.float32),
        ] + [pltpu.VMEM((NB,), jnp.int32)] * NSTREAM,
    )
    return f(km, fn)


def _sigmoid_body(x_ref, m_ref, o_ref):
    t = (x_ref[...] - m_ref[0, 0]) / 0.7
    o_ref[...] = 1.0 / (1.0 + jnp.exp(-t))


@jax.jit
def _tc_sigmoid(fn, mean):
    return pl.pallas_call(
        _sigmoid_body,
        out_shape=jax.ShapeDtypeStruct((L, N), jnp.float32),
        in_specs=[
            pl.BlockSpec((L, N), lambda: (0, 0)),
            pl.BlockSpec(memory_space=pltpu.SMEM),
        ],
        out_specs=pl.BlockSpec((L, N), lambda: (0, 0)),
    )(fn, jnp.reshape(mean, (1, 1)))


def kernel(k_masks, fn, mean):
    ori_masks = _sc_sort_scatter(k_masks, fn)
    cout = _tc_sigmoid(fn, mean)
    return ori_masks, cout
